# baseline (speedup 1.0000x reference)
import functools
import math

import jax
import jax.numpy as jnp
from jax import lax
from jax.experimental import pallas as pl
from jax.experimental.pallas import tpu as pltpu
from jax.experimental.pallas import tpu_sc as plsc

D_MODEL = 64
SCALE = math.sqrt(D_MODEL)
NC, NS = 2, 16
NW = NC * NS
LANES = 16
C = 256
NBUF = 2
SSTR = 65

B0, B1 = 4096, 200
TOTAL = B0 * B1
CH_PER_W = TOTAL // C // NW
QPB = B0 // C
GPC = C // 128


@jax.jit
def _emb_lookup(x_t, table):
    mesh = plsc.VectorSubcoreMesh(core_axis_name="c", subcore_axis_name="s")

    @functools.partial(
        pl.kernel,
        out_type=jax.ShapeDtypeStruct((B1, 8, B0 // 128, 8, 128), jnp.float32),
        mesh=mesh,
        scratch_types=(
            [pltpu.VMEM((C,), jnp.int32) for _ in range(NBUF)]
            + [pltpu.VMEM((C, D_MODEL), jnp.float32) for _ in range(NBUF)]
            + [pltpu.VMEM((C * SSTR,), jnp.float32) for _ in range(NBUF)]
            + [pltpu.VMEM((8, GPC, 8, 128), jnp.float32) for _ in range(NBUF)]
            + [pltpu.SemaphoreType.DMA for _ in range(2 * NBUF)]
        ),
        compiler_params=pltpu.CompilerParams(
            use_tc_tiling_on_sc=False, needs_layout_passes=False
        ),
    )
    def body(x_hbm, table_hbm, out_hbm, *scratch):
        idx = scratch[:NBUF]
        rows = scratch[NBUF:2 * NBUF]
        stg = scratch[2 * NBUF:3 * NBUF]
        tbuf = scratch[3 * NBUF:4 * NBUF]
        gsem = scratch[4 * NBUF:5 * NBUF]
        osem = scratch[5 * NBUF:]

        wid = lax.axis_index("s") * NC + lax.axis_index("c")
        chunk0 = wid * CH_PER_W
        iota = lax.iota(jnp.int32, LANES)

        def start_gather(b, gc):
            start = pl.multiple_of(gc * C, C)
            pltpu.sync_copy(x_hbm.at[pl.ds(start, C)], idx[b])
            pltpu.async_copy(table_hbm.at[idx[b]], rows[b], gsem[b])

        def wait_gather(b):
            pltpu.make_async_copy(table_hbm.at[idx[b]], rows[b], gsem[b]).wait()

        def reformat(b):
            def s1(r, carry):
                for jj in range(D_MODEL // LANES):
                    stg[b][pl.ds(r * SSTR + jj * LANES, LANES)] = (
                        rows[b][r, pl.ds(jj * LANES, LANES)]
                    )
                return carry

            lax.fori_loop(0, C, s1, 0, unroll=2)

            def s2(j, carry):
                g2 = j // 8
                j16 = j % 8
                addr0 = (j * LANES + iota) * SSTR
                for d in range(D_MODEL):
                    v = plsc.load_gather(stg[b], [addr0 + d])
                    tbuf[b][d // 8, g2, d % 8, pl.ds(j16 * LANES, LANES)] = (
                        v * SCALE
                    )
                return carry

            lax.fori_loop(0, C // LANES, s2, 0)

        def write_out(b, b1, q):
            pltpu.async_copy(
                tbuf[b], out_hbm.at[b1, :, pl.ds(GPC * q, GPC)], osem[b]
            )

        def wait_writes(b, b1, q):
            pltpu.make_async_copy(
                tbuf[b], out_hbm.at[b1, :, pl.ds(GPC * q, GPC)], osem[b]
            ).wait()

        def process(b, gc, do_wait_writes):
            b1 = gc // QPB
            q = gc % QPB
            wait_gather(b)
            if do_wait_writes:
                wait_writes(b, b1, q)
            reformat(b)
            write_out(b, b1, q)

        for b in range(NBUF):
            start_gather(b, chunk0 + b)
        for b in range(NBUF):
            process(b, chunk0 + b, do_wait_writes=False)
            start_gather(b, chunk0 + b + NBUF)

        def main(i, carry):
            for b in range(NBUF):
                gc = chunk0 + i * NBUF + b
                process(b, gc, do_wait_writes=True)
                start_gather(b, gc + NBUF)
            return carry

        lax.fori_loop(1, CH_PER_W // NBUF - 1, main, 0)

        for b in range(NBUF):
            gc = chunk0 + CH_PER_W - NBUF + b
            process(b, gc, do_wait_writes=True)
        for b in range(NBUF):
            gc = chunk0 + CH_PER_W - NBUF + b
            wait_writes(b, gc // QPB, gc % QPB)

    return body(x_t, table)


def kernel(x, table):
    x_t = x.T.reshape(TOTAL)
    out5 = _emb_lookup(x_t, table)
    return out5.transpose(2, 4, 0, 1, 3).reshape(B0, B1, D_MODEL)
